# Initial kernel scaffold; baseline (speedup 1.0000x reference)
import jax
import jax.numpy as jnp
from jax.experimental import pallas as pl


def kernel(speaker, x, edge_index, edge_norm, edge_type, seq_lengths, umask, w1, w2, Wl, bl, Ws, bs):
    raise NotImplementedError("write your pallas kernel here")


# speedup vs baseline: 12.2552x; 12.2552x over previous
import jax
import jax.numpy as jnp
from jax import lax
from jax.experimental import pallas as pl
from jax.experimental.pallas import tpu as pltpu
from jax.experimental.pallas import tpu_sc as plsc

N = 10000
E = 320000
H = 128
NCORE = 2
NSUB = 16
NW = NCORE * NSUB
K = 128
CH = 80
E_PAD = NW * K * CH
N_PAD = 10240
STRIPE = N_PAD // NSUB
BN = 1000

_MESH = dict(core_axis_name="c", subcore_axis_name="s",
             num_cores=NCORE, num_subcores=NSUB)


def _make_sc_agg():
    def body(ycat, adj1, dst1, norm1, zrow,
             acc_hbm, acc_sh, adj_v, dst_v, norm_v, rows_v, gsem):
        core = lax.axis_index("c")
        sub = lax.axis_index("s")
        wid = core * NSUB + sub
        base = wid * CH * K
        rs = pl.ds(sub * STRIPE, STRIPE)
        pltpu.sync_copy(zrow.at[rs], acc_sh.at[rs])
        plsc.subcore_barrier()

        def chunk(c, carry):
            off = base + c * K
            pltpu.sync_copy(adj1.at[pl.ds(off, K)], adj_v)
            pltpu.sync_copy(dst1.at[pl.ds(off, K)], dst_v)
            pltpu.sync_copy(norm1.at[pl.ds(off, K)], norm_v)
            pltpu.async_copy(ycat.at[adj_v], rows_v, gsem).wait()

            def group(g, carry2):
                nv = norm_v[pl.ds(g * 16, 16)]
                for e16 in range(16):
                    n = nv[e16]
                    row = g * 16 + e16
                    for j in range(H // 16):
                        sl = pl.ds(j * 16, 16)
                        rows_v[row, sl] = rows_v[row, sl] * n
                return carry2
            lax.fori_loop(0, K // 16, group, 0)
            pltpu.sync_copy(rows_v, acc_sh.at[dst_v], add=True)
            return carry
        lax.fori_loop(0, CH, chunk, 0)

        plsc.subcore_barrier()
        pltpu.sync_copy(acc_sh.at[rs], acc_hbm.at[core, rs])

    return pl.kernel(
        body,
        out_type=jax.ShapeDtypeStruct((NCORE, N_PAD, H), jnp.float32),
        mesh=plsc.VectorSubcoreMesh(**_MESH),
        scratch_types=[
            pltpu.VMEM_SHARED((N_PAD, H), jnp.float32),
            pltpu.VMEM((K,), jnp.int32),
            pltpu.VMEM((K,), jnp.int32),
            pltpu.VMEM((K,), jnp.float32),
            pltpu.VMEM((K, H), jnp.float32),
            pltpu.SemaphoreType.DMA,
        ],
    )


def _make_sc_count():
    def body(dst1, zrow, ones_hbm, cnt_hbm, cnt_sh, dst_v, ones_v):
        core = lax.axis_index("c")
        sub = lax.axis_index("s")
        wid = core * NSUB + sub
        base = wid * CH * K
        rs = pl.ds(sub * STRIPE, STRIPE)
        pltpu.sync_copy(zrow.at[rs], cnt_sh.at[rs])
        pltpu.sync_copy(ones_hbm, ones_v)
        plsc.subcore_barrier()

        def chunk(c, carry):
            off = base + c * K
            pltpu.sync_copy(dst1.at[pl.ds(off, K)], dst_v)
            pltpu.sync_copy(ones_v, cnt_sh.at[dst_v], add=True)
            return carry
        lax.fori_loop(0, CH, chunk, 0)

        plsc.subcore_barrier()
        pltpu.sync_copy(cnt_sh.at[rs], cnt_hbm.at[core, rs])

    return pl.kernel(
        body,
        out_type=jax.ShapeDtypeStruct((NCORE, N_PAD, H), jnp.float32),
        mesh=plsc.VectorSubcoreMesh(**_MESH),
        scratch_types=[
            pltpu.VMEM_SHARED((N_PAD, H), jnp.float32),
            pltpu.VMEM((K,), jnp.int32),
            pltpu.VMEM((K, H), jnp.float32),
        ],
    )


def _mm2_body(x_ref, w_ref, o_ref):
    o_ref[...] = jnp.dot(x_ref[...], w_ref[0],
                         preferred_element_type=jnp.float32)


def _mm2(x, w):
    return pl.pallas_call(
        _mm2_body,
        grid=(2, N // BN),
        in_specs=[
            pl.BlockSpec((BN, H), lambda r, i: (i, 0)),
            pl.BlockSpec((1, H, H), lambda r, i: (r, 0, 0)),
        ],
        out_specs=pl.BlockSpec((BN, H), lambda r, i: (r * (N // BN) + i, 0)),
        out_shape=jax.ShapeDtypeStruct((2 * N, H), jnp.float32),
    )(x, w)


def _combine_mm2_body(acc_ref, cnt_ref, w_ref, o_ref, inv_ref):
    ssum = acc_ref[0] + acc_ref[1]
    cnt = cnt_ref[0, :, 0] + cnt_ref[1, :, 0]
    inv = 1.0 / jnp.maximum(cnt, 1.0)
    out1 = jax.nn.sigmoid(ssum * inv[:, None])
    o_ref[...] = jnp.dot(out1, w_ref[0], preferred_element_type=jnp.float32)
    inv_ref[...] = inv[:, None] * jnp.ones((1, 8), jnp.float32)


def _combine_mm2(acc, cnt, w):
    return pl.pallas_call(
        _combine_mm2_body,
        grid=(2, N // BN),
        in_specs=[
            pl.BlockSpec((2, BN, H), lambda r, i: (0, i, 0)),
            pl.BlockSpec((2, BN, H), lambda r, i: (0, i, 0)),
            pl.BlockSpec((1, H, H), lambda r, i: (r, 0, 0)),
        ],
        out_specs=[
            pl.BlockSpec((BN, H), lambda r, i: (r * (N // BN) + i, 0)),
            pl.BlockSpec((BN, 8), lambda r, i: (i, 0)),
        ],
        out_shape=[
            jax.ShapeDtypeStruct((2 * N, H), jnp.float32),
            jax.ShapeDtypeStruct((N, 8), jnp.float32),
        ],
    )(acc, cnt, w)


def _final_body(acc_ref, inv_ref, x_ref, wlt_ref, wlb_ref, bl_ref,
                ws_ref, bs_ref, out2_ref, lp_ref):
    inv = inv_ref[:, 0]
    out2 = (acc_ref[0] + acc_ref[1]) * inv[:, None]
    out2_ref[...] = out2
    x = x_ref[...]
    hidden = jnp.dot(x, wlt_ref[...], preferred_element_type=jnp.float32)
    hidden += jnp.dot(out2, wlb_ref[...], preferred_element_type=jnp.float32)
    hidden = jax.nn.relu(hidden + bl_ref[...])
    logits = jnp.dot(hidden, ws_ref[...],
                     preferred_element_type=jnp.float32) + bs_ref[...]
    m = jnp.max(logits, axis=1, keepdims=True)
    z = logits - m
    lse = jnp.log(jnp.sum(jnp.exp(z), axis=1, keepdims=True))
    lp_ref[...] = z - lse


def _final(acc, inv8, x, wlt, wlb, bl, ws8, bs8):
    return pl.pallas_call(
        _final_body,
        grid=(N // BN,),
        in_specs=[
            pl.BlockSpec((2, BN, H), lambda i: (0, i, 0)),
            pl.BlockSpec((BN, 8), lambda i: (i, 0)),
            pl.BlockSpec((BN, H), lambda i: (i, 0)),
            pl.BlockSpec((H, H), lambda i: (0, 0)),
            pl.BlockSpec((H, H), lambda i: (0, 0)),
            pl.BlockSpec((1, H), lambda i: (0, 0)),
            pl.BlockSpec((H, 8), lambda i: (0, 0)),
            pl.BlockSpec((1, 8), lambda i: (0, 0)),
        ],
        out_specs=[
            pl.BlockSpec((BN, H), lambda i: (i, 0)),
            pl.BlockSpec((BN, 8), lambda i: (i, 0)),
        ],
        out_shape=[
            jax.ShapeDtypeStruct((N, H), jnp.float32),
            jax.ShapeDtypeStruct((N, 8), jnp.float32),
        ],
    )(acc, inv8, x, wlt, wlb, bl, ws8, bs8)


_sc_agg = _make_sc_agg()
_sc_count = _make_sc_count()


@jax.jit
def kernel(speaker, x, edge_index, edge_norm, edge_type, seq_lengths, umask,
           w1, w2, Wl, bl, Ws, bs):
    dst = edge_index[0]
    src = edge_index[1]
    adj = edge_type * N + src
    norm2 = edge_norm * 2.0

    pad = E_PAD - E
    adj1 = jnp.concatenate([adj, jnp.zeros((pad,), jnp.int32)])
    dst1 = jnp.concatenate([dst, jnp.full((pad,), N, jnp.int32)])
    norm1 = jnp.concatenate([norm2, jnp.zeros((pad,), jnp.float32)])

    zrow = jnp.zeros((N_PAD, H), jnp.float32)
    ones_rows = jnp.ones((K, H), jnp.float32)

    cnt = _sc_count(dst1, zrow, ones_rows)
    ycat1 = _mm2(x, w1)
    acc1 = _sc_agg(ycat1, adj1, dst1, norm1, zrow)
    ycat2, inv8 = _combine_mm2(acc1, cnt, w2)
    acc2 = _sc_agg(ycat2, adj1, dst1, norm1, zrow)
    wlt = Wl[:H]
    wlb = Wl[H:]
    ws8 = jnp.zeros((H, 8), jnp.float32).at[:, :7].set(Ws)
    bs8 = jnp.full((1, 8), -1e30, jnp.float32).at[0, :7].set(bs)
    out2, lp8 = _final(acc2, inv8, x, wlt, wlb, bl.reshape(1, H), ws8, bs8)

    emotions = jnp.concatenate([x, out2], axis=1)
    return (lp8[:, :7], x, emotions)


# speedup vs baseline: 16.1321x; 1.3163x over previous
import jax
import jax.numpy as jnp
from jax import lax
from jax.experimental import pallas as pl
from jax.experimental.pallas import tpu as pltpu
from jax.experimental.pallas import tpu_sc as plsc

N = 10000
E = 320000
H = 128
NCORE = 2
NSUB = 16
NW = NCORE * NSUB
K = 64
CH = 160
NBUF = 4
RING = 8
E_PAD = NW * K * CH
N_PAD = 10240
STRIPE = N_PAD // NSUB
BN = 1000

_MESH = dict(core_axis_name="c", subcore_axis_name="s",
             num_cores=NCORE, num_subcores=NSUB)


def _make_sc_agg():
    def body(ycat, adj1, dst1, norm1, zrow, acc_hbm,
             acc_sh, rows, abuf, dbuf, nbuf, gsm, ssm, asm, dsm, nsm):
        core = lax.axis_index("c")
        sub = lax.axis_index("s")
        wid = core * NSUB + sub
        base = wid * CH * K
        rs = pl.ds(sub * STRIPE, STRIPE)
        pltpu.sync_copy(zrow.at[rs], acc_sh.at[rs])
        plsc.subcore_barrier()

        def issue_edata(c, s):
            off = base + c * K
            pltpu.async_copy(adj1.at[pl.ds(off, K)], abuf[s], asm[s])
            pltpu.async_copy(dst1.at[pl.ds(off, K)], dbuf[s], dsm[s])
            pltpu.async_copy(norm1.at[pl.ds(off, K)], nbuf[s], nsm[s])

        def wait_edata(c, s):
            off = base + c * K
            pltpu.make_async_copy(adj1.at[pl.ds(off, K)], abuf[s],
                                  asm[s]).wait()
            pltpu.make_async_copy(dst1.at[pl.ds(off, K)], dbuf[s],
                                  dsm[s]).wait()
            pltpu.make_async_copy(norm1.at[pl.ds(off, K)], nbuf[s],
                                  nsm[s]).wait()

        def issue_gather(s, p):
            pltpu.async_copy(ycat.at[abuf[s]], rows[p], gsm[p])

        def wait_gather(s, p):
            pltpu.make_async_copy(ycat.at[abuf[s]], rows[p], gsm[p]).wait()

        def issue_scatter(s, p):
            pltpu.async_copy(rows[p], acc_sh.at[dbuf[s]], ssm[p], add=True)

        def wait_scatter(s, p):
            pltpu.make_async_copy(
                rows[p], acc_sh.at[dbuf[s]], ssm[p]).wait()

        def scale(u):
            def group(g, carry2):
                nv = nbuf[u][pl.ds(g * 16, 16)]
                for e16 in range(16):
                    n = nv[e16]
                    row = g * 16 + e16
                    for j in range(H // 16):
                        sl = pl.ds(j * 16, 16)
                        rows[u][row, sl] = rows[u][row, sl] * n
                return carry2
            lax.fori_loop(0, K // 16, group, 0)

        issue_edata(0, 0)
        wait_edata(0, 0)
        issue_gather(0, 0)

        def pair(t, carry):
            for u in range(2):
                c = t * 2 + u
                issue_edata(c + 1, 1 - u)
                wait_edata(c + 1, 1 - u)
                issue_gather(1 - u, 1 - u)
                wait_gather(u, u)
                scale(u)
                issue_scatter(u, u)
                wait_scatter(u, u)
            return carry
        lax.fori_loop(0, CH // 2 - 1, pair, 0)

        issue_edata(CH - 1, 1)
        wait_edata(CH - 1, 1)
        issue_gather(1, 1)
        wait_gather(0, 0)
        scale(0)
        issue_scatter(0, 0)
        wait_scatter(0, 0)
        wait_gather(1, 1)
        scale(1)
        issue_scatter(1, 1)
        wait_scatter(1, 1)

        plsc.subcore_barrier()
        pltpu.sync_copy(acc_sh.at[rs], acc_hbm.at[core, rs])

    return pl.kernel(
        body,
        out_type=jax.ShapeDtypeStruct((NCORE, N_PAD, H), jnp.float32),
        mesh=plsc.VectorSubcoreMesh(**_MESH),
        scratch_types=[
            pltpu.VMEM_SHARED((N_PAD, H), jnp.float32),
            [pltpu.VMEM((K, H), jnp.float32) for _ in range(NBUF)],
            [pltpu.VMEM((K,), jnp.int32) for _ in range(RING)],
            [pltpu.VMEM((K,), jnp.int32) for _ in range(RING)],
            [pltpu.VMEM((K,), jnp.float32) for _ in range(RING)],
            [pltpu.SemaphoreType.DMA for _ in range(NBUF)],
            [pltpu.SemaphoreType.DMA for _ in range(NBUF)],
            [pltpu.SemaphoreType.DMA for _ in range(RING)],
            [pltpu.SemaphoreType.DMA for _ in range(RING)],
            [pltpu.SemaphoreType.DMA for _ in range(RING)],
        ],
    )


def _make_sc_count():
    def body(dst1, zrow, ones_hbm, cnt_hbm, cnt_sh, dst_v, ones_v):
        core = lax.axis_index("c")
        sub = lax.axis_index("s")
        wid = core * NSUB + sub
        base = wid * CH * K
        rs = pl.ds(sub * STRIPE, STRIPE)
        pltpu.sync_copy(zrow.at[rs], cnt_sh.at[rs])
        pltpu.sync_copy(ones_hbm, ones_v)
        plsc.subcore_barrier()

        def chunk(c, carry):
            off = base + c * K
            pltpu.sync_copy(dst1.at[pl.ds(off, K)], dst_v)
            pltpu.sync_copy(ones_v, cnt_sh.at[dst_v], add=True)
            return carry
        lax.fori_loop(0, CH, chunk, 0)

        plsc.subcore_barrier()
        pltpu.sync_copy(cnt_sh.at[rs], cnt_hbm.at[core, rs])

    return pl.kernel(
        body,
        out_type=jax.ShapeDtypeStruct((NCORE, N_PAD, H), jnp.float32),
        mesh=plsc.VectorSubcoreMesh(**_MESH),
        scratch_types=[
            pltpu.VMEM_SHARED((N_PAD, H), jnp.float32),
            pltpu.VMEM((K,), jnp.int32),
            pltpu.VMEM((K, H), jnp.float32),
        ],
    )


def _mm2_body(x_ref, w_ref, o_ref):
    o_ref[...] = jnp.dot(x_ref[...], w_ref[0],
                         preferred_element_type=jnp.float32)


def _mm2(x, w):
    return pl.pallas_call(
        _mm2_body,
        grid=(2, N // BN),
        in_specs=[
            pl.BlockSpec((BN, H), lambda r, i: (i, 0)),
            pl.BlockSpec((1, H, H), lambda r, i: (r, 0, 0)),
        ],
        out_specs=pl.BlockSpec((BN, H), lambda r, i: (r * (N // BN) + i, 0)),
        out_shape=jax.ShapeDtypeStruct((2 * N, H), jnp.float32),
    )(x, w)


def _combine_mm2_body(acc_ref, cnt_ref, w_ref, o_ref, inv_ref):
    ssum = acc_ref[0] + acc_ref[1]
    cnt = cnt_ref[0, :, 0] + cnt_ref[1, :, 0]
    inv = 1.0 / jnp.maximum(cnt, 1.0)
    out1 = jax.nn.sigmoid(ssum * inv[:, None])
    o_ref[...] = jnp.dot(out1, w_ref[0], preferred_element_type=jnp.float32)
    inv_ref[...] = inv[:, None] * jnp.ones((1, 8), jnp.float32)


def _combine_mm2(acc, cnt, w):
    return pl.pallas_call(
        _combine_mm2_body,
        grid=(2, N // BN),
        in_specs=[
            pl.BlockSpec((2, BN, H), lambda r, i: (0, i, 0)),
            pl.BlockSpec((2, BN, H), lambda r, i: (0, i, 0)),
            pl.BlockSpec((1, H, H), lambda r, i: (r, 0, 0)),
        ],
        out_specs=[
            pl.BlockSpec((BN, H), lambda r, i: (r * (N // BN) + i, 0)),
            pl.BlockSpec((BN, 8), lambda r, i: (i, 0)),
        ],
        out_shape=[
            jax.ShapeDtypeStruct((2 * N, H), jnp.float32),
            jax.ShapeDtypeStruct((N, 8), jnp.float32),
        ],
    )(acc, cnt, w)


def _final_body(acc_ref, inv_ref, x_ref, wlt_ref, wlb_ref, bl_ref,
                ws_ref, bs_ref, out2_ref, lp_ref):
    inv = inv_ref[:, 0]
    out2 = (acc_ref[0] + acc_ref[1]) * inv[:, None]
    out2_ref[...] = out2
    x = x_ref[...]
    hidden = jnp.dot(x, wlt_ref[...], preferred_element_type=jnp.float32)
    hidden += jnp.dot(out2, wlb_ref[...], preferred_element_type=jnp.float32)
    hidden = jax.nn.relu(hidden + bl_ref[...])
    logits = jnp.dot(hidden, ws_ref[...],
                     preferred_element_type=jnp.float32) + bs_ref[...]
    m = jnp.max(logits, axis=1, keepdims=True)
    z = logits - m
    lse = jnp.log(jnp.sum(jnp.exp(z), axis=1, keepdims=True))
    lp_ref[...] = z - lse


def _final(acc, inv8, x, wlt, wlb, bl, ws8, bs8):
    return pl.pallas_call(
        _final_body,
        grid=(N // BN,),
        in_specs=[
            pl.BlockSpec((2, BN, H), lambda i: (0, i, 0)),
            pl.BlockSpec((BN, 8), lambda i: (i, 0)),
            pl.BlockSpec((BN, H), lambda i: (i, 0)),
            pl.BlockSpec((H, H), lambda i: (0, 0)),
            pl.BlockSpec((H, H), lambda i: (0, 0)),
            pl.BlockSpec((1, H), lambda i: (0, 0)),
            pl.BlockSpec((H, 8), lambda i: (0, 0)),
            pl.BlockSpec((1, 8), lambda i: (0, 0)),
        ],
        out_specs=[
            pl.BlockSpec((BN, H), lambda i: (i, 0)),
            pl.BlockSpec((BN, 8), lambda i: (i, 0)),
        ],
        out_shape=[
            jax.ShapeDtypeStruct((N, H), jnp.float32),
            jax.ShapeDtypeStruct((N, 8), jnp.float32),
        ],
    )(acc, inv8, x, wlt, wlb, bl, ws8, bs8)


_sc_agg = _make_sc_agg()
_sc_count = _make_sc_count()


@jax.jit
def kernel(speaker, x, edge_index, edge_norm, edge_type, seq_lengths, umask,
           w1, w2, Wl, bl, Ws, bs):
    dst = edge_index[0]
    src = edge_index[1]
    adj = edge_type * N + src
    norm2 = edge_norm * 2.0

    pad = E_PAD - E
    adj1 = jnp.concatenate([adj, jnp.zeros((pad,), jnp.int32)])
    dst1 = jnp.concatenate([dst, jnp.full((pad,), N, jnp.int32)])
    norm1 = jnp.concatenate([norm2, jnp.zeros((pad,), jnp.float32)])

    zrow = jnp.zeros((N_PAD, H), jnp.float32)
    ones_rows = jnp.ones((K, H), jnp.float32)

    cnt = _sc_count(dst1, zrow, ones_rows)
    ycat1 = _mm2(x, w1)
    acc1 = _sc_agg(ycat1, adj1, dst1, norm1, zrow)
    ycat2, inv8 = _combine_mm2(acc1, cnt, w2)
    acc2 = _sc_agg(ycat2, adj1, dst1, norm1, zrow)
    wlt = Wl[:H]
    wlb = Wl[H:]
    ws8 = jnp.zeros((H, 8), jnp.float32).at[:, :7].set(Ws)
    bs8 = jnp.full((1, 8), -1e30, jnp.float32).at[0, :7].set(bs)
    out2, lp8 = _final(acc2, inv8, x, wlt, wlb, bl.reshape(1, H), ws8, bs8)

    emotions = jnp.concatenate([x, out2], axis=1)
    return (lp8[:, :7], x, emotions)


# speedup vs baseline: 16.4312x; 1.0185x over previous
import jax
import jax.numpy as jnp
from jax import lax
from jax.experimental import pallas as pl
from jax.experimental.pallas import tpu as pltpu
from jax.experimental.pallas import tpu_sc as plsc

N = 10000
E = 320000
H = 128
NCORE = 2
NSUB = 16
NW = NCORE * NSUB
K = 64
CH = 160
NBUF = 4
RING = 8
E_PAD = NW * K * CH
N_PAD = 10240
STRIPE = N_PAD // NSUB
BN = 1000

_MESH = dict(core_axis_name="c", subcore_axis_name="s",
             num_cores=NCORE, num_subcores=NSUB)


def _make_sc_agg():
    def body(ycat, adj1, dst1, norm1, zrow, acc_hbm,
             acc_sh, rows, abuf, dbuf, nbuf, gsm, ssm, asm, dsm, nsm):
        core = lax.axis_index("c")
        sub = lax.axis_index("s")
        wid = core * NSUB + sub
        base = wid * CH * K
        rs = pl.ds(sub * STRIPE, STRIPE)
        pltpu.sync_copy(zrow.at[rs], acc_sh.at[rs])
        plsc.subcore_barrier()

        def issue_edata(c, s):
            off = base + c * K
            pltpu.async_copy(adj1.at[pl.ds(off, K)], abuf[s], asm[s])
            pltpu.async_copy(dst1.at[pl.ds(off, K)], dbuf[s], dsm[s])
            pltpu.async_copy(norm1.at[pl.ds(off, K)], nbuf[s], nsm[s])

        def wait_edata(c, s):
            off = base + c * K
            pltpu.make_async_copy(adj1.at[pl.ds(off, K)], abuf[s],
                                  asm[s]).wait()
            pltpu.make_async_copy(dst1.at[pl.ds(off, K)], dbuf[s],
                                  dsm[s]).wait()
            pltpu.make_async_copy(norm1.at[pl.ds(off, K)], nbuf[s],
                                  nsm[s]).wait()

        def issue_gather(s, p):
            pltpu.async_copy(ycat.at[abuf[s]], rows[p], gsm[p])

        def wait_gather(s, p):
            pltpu.make_async_copy(ycat.at[abuf[s]], rows[p], gsm[p]).wait()

        def issue_scatter(s, p):
            pltpu.async_copy(rows[p], acc_sh.at[dbuf[s]], ssm[p], add=True)

        def wait_scatter(s, p):
            pltpu.make_async_copy(
                rows[p], acc_sh.at[dbuf[s]], ssm[p]).wait()

        def scale(u):
            def group(g, carry2):
                nv = nbuf[u][pl.ds(g * 16, 16)]
                for e16 in range(16):
                    n = nv[e16]
                    row = g * 16 + e16
                    for j in range(H // 16):
                        sl = pl.ds(j * 16, 16)
                        rows[u][row, sl] = rows[u][row, sl] * n
                return carry2
            lax.fori_loop(0, K // 16, group, 0)

        def slot(c, u, e2=True, g1=True, ws=True):
            if e2:
                issue_edata(c + 2, (u + 2) % 4)
            if g1:
                wait_edata(c + 1, (u + 1) % 4)
                issue_gather((u + 1) % 4, (u + 1) % 4)
            wait_gather(u, u)
            scale(u)
            issue_scatter(u, u)
            if ws:
                wait_scatter((u + 3) % 4, (u + 3) % 4)

        issue_edata(0, 0)
        issue_edata(1, 1)
        wait_edata(0, 0)
        issue_gather(0, 0)
        slot(0, 0, ws=False)
        for u in range(1, 4):
            slot(u, u)

        def block(t, carry):
            for u in range(4):
                slot(t * 4 + u, u)
            return carry
        lax.fori_loop(1, CH // 4 - 1, block, 0)

        slot(CH - 4, 0)
        slot(CH - 3, 1)
        slot(CH - 2, 2, e2=False)
        slot(CH - 1, 3, e2=False, g1=False)
        wait_scatter(3, 3)

        plsc.subcore_barrier()
        pltpu.sync_copy(acc_sh.at[rs], acc_hbm.at[core, rs])

    return pl.kernel(
        body,
        out_type=jax.ShapeDtypeStruct((NCORE, N_PAD, H), jnp.float32),
        mesh=plsc.VectorSubcoreMesh(**_MESH),
        scratch_types=[
            pltpu.VMEM_SHARED((N_PAD, H), jnp.float32),
            [pltpu.VMEM((K, H), jnp.float32) for _ in range(NBUF)],
            [pltpu.VMEM((K,), jnp.int32) for _ in range(RING)],
            [pltpu.VMEM((K,), jnp.int32) for _ in range(RING)],
            [pltpu.VMEM((K,), jnp.float32) for _ in range(RING)],
            [pltpu.SemaphoreType.DMA for _ in range(NBUF)],
            [pltpu.SemaphoreType.DMA for _ in range(NBUF)],
            [pltpu.SemaphoreType.DMA for _ in range(RING)],
            [pltpu.SemaphoreType.DMA for _ in range(RING)],
            [pltpu.SemaphoreType.DMA for _ in range(RING)],
        ],
    )


def _make_sc_count():
    def body(dst1, zrow, ones_hbm, cnt_hbm, cnt_sh, dst_v, ones_v):
        core = lax.axis_index("c")
        sub = lax.axis_index("s")
        wid = core * NSUB + sub
        base = wid * CH * K
        rs = pl.ds(sub * STRIPE, STRIPE)
        pltpu.sync_copy(zrow.at[rs], cnt_sh.at[rs])
        pltpu.sync_copy(ones_hbm, ones_v)
        plsc.subcore_barrier()

        def chunk(c, carry):
            off = base + c * K
            pltpu.sync_copy(dst1.at[pl.ds(off, K)], dst_v)
            pltpu.sync_copy(ones_v, cnt_sh.at[dst_v], add=True)
            return carry
        lax.fori_loop(0, CH, chunk, 0)

        plsc.subcore_barrier()
        pltpu.sync_copy(cnt_sh.at[rs], cnt_hbm.at[core, rs])

    return pl.kernel(
        body,
        out_type=jax.ShapeDtypeStruct((NCORE, N_PAD, H), jnp.float32),
        mesh=plsc.VectorSubcoreMesh(**_MESH),
        scratch_types=[
            pltpu.VMEM_SHARED((N_PAD, H), jnp.float32),
            pltpu.VMEM((K,), jnp.int32),
            pltpu.VMEM((K, H), jnp.float32),
        ],
    )


def _mm2_body(x_ref, w_ref, o_ref):
    o_ref[...] = jnp.dot(x_ref[...], w_ref[0],
                         preferred_element_type=jnp.float32)


def _mm2(x, w):
    return pl.pallas_call(
        _mm2_body,
        grid=(2, N // BN),
        in_specs=[
            pl.BlockSpec((BN, H), lambda r, i: (i, 0)),
            pl.BlockSpec((1, H, H), lambda r, i: (r, 0, 0)),
        ],
        out_specs=pl.BlockSpec((BN, H), lambda r, i: (r * (N // BN) + i, 0)),
        out_shape=jax.ShapeDtypeStruct((2 * N, H), jnp.float32),
    )(x, w)


def _combine_mm2_body(acc_ref, cnt_ref, w_ref, o_ref, inv_ref):
    ssum = acc_ref[0] + acc_ref[1]
    cnt = cnt_ref[0, :, 0] + cnt_ref[1, :, 0]
    inv = 1.0 / jnp.maximum(cnt, 1.0)
    out1 = jax.nn.sigmoid(ssum * inv[:, None])
    o_ref[...] = jnp.dot(out1, w_ref[0], preferred_element_type=jnp.float32)
    inv_ref[...] = inv[:, None] * jnp.ones((1, 8), jnp.float32)


def _combine_mm2(acc, cnt, w):
    return pl.pallas_call(
        _combine_mm2_body,
        grid=(2, N // BN),
        in_specs=[
            pl.BlockSpec((2, BN, H), lambda r, i: (0, i, 0)),
            pl.BlockSpec((2, BN, H), lambda r, i: (0, i, 0)),
            pl.BlockSpec((1, H, H), lambda r, i: (r, 0, 0)),
        ],
        out_specs=[
            pl.BlockSpec((BN, H), lambda r, i: (r * (N // BN) + i, 0)),
            pl.BlockSpec((BN, 8), lambda r, i: (i, 0)),
        ],
        out_shape=[
            jax.ShapeDtypeStruct((2 * N, H), jnp.float32),
            jax.ShapeDtypeStruct((N, 8), jnp.float32),
        ],
    )(acc, cnt, w)


def _final_body(acc_ref, inv_ref, x_ref, wlt_ref, wlb_ref, bl_ref,
                ws_ref, bs_ref, out2_ref, lp_ref):
    inv = inv_ref[:, 0]
    out2 = (acc_ref[0] + acc_ref[1]) * inv[:, None]
    out2_ref[...] = out2
    x = x_ref[...]
    hidden = jnp.dot(x, wlt_ref[...], preferred_element_type=jnp.float32)
    hidden += jnp.dot(out2, wlb_ref[...], preferred_element_type=jnp.float32)
    hidden = jax.nn.relu(hidden + bl_ref[...])
    logits = jnp.dot(hidden, ws_ref[...],
                     preferred_element_type=jnp.float32) + bs_ref[...]
    m = jnp.max(logits, axis=1, keepdims=True)
    z = logits - m
    lse = jnp.log(jnp.sum(jnp.exp(z), axis=1, keepdims=True))
    lp_ref[...] = z - lse


def _final(acc, inv8, x, wlt, wlb, bl, ws8, bs8):
    return pl.pallas_call(
        _final_body,
        grid=(N // BN,),
        in_specs=[
            pl.BlockSpec((2, BN, H), lambda i: (0, i, 0)),
            pl.BlockSpec((BN, 8), lambda i: (i, 0)),
            pl.BlockSpec((BN, H), lambda i: (i, 0)),
            pl.BlockSpec((H, H), lambda i: (0, 0)),
            pl.BlockSpec((H, H), lambda i: (0, 0)),
            pl.BlockSpec((1, H), lambda i: (0, 0)),
            pl.BlockSpec((H, 8), lambda i: (0, 0)),
            pl.BlockSpec((1, 8), lambda i: (0, 0)),
        ],
        out_specs=[
            pl.BlockSpec((BN, H), lambda i: (i, 0)),
            pl.BlockSpec((BN, 8), lambda i: (i, 0)),
        ],
        out_shape=[
            jax.ShapeDtypeStruct((N, H), jnp.float32),
            jax.ShapeDtypeStruct((N, 8), jnp.float32),
        ],
    )(acc, inv8, x, wlt, wlb, bl, ws8, bs8)


_sc_agg = _make_sc_agg()
_sc_count = _make_sc_count()


@jax.jit
def kernel(speaker, x, edge_index, edge_norm, edge_type, seq_lengths, umask,
           w1, w2, Wl, bl, Ws, bs):
    dst = edge_index[0]
    src = edge_index[1]
    adj = edge_type * N + src
    norm2 = edge_norm * 2.0

    pad = E_PAD - E
    adj1 = jnp.concatenate([adj, jnp.zeros((pad,), jnp.int32)])
    dst1 = jnp.concatenate([dst, jnp.full((pad,), N, jnp.int32)])
    norm1 = jnp.concatenate([norm2, jnp.zeros((pad,), jnp.float32)])

    zrow = jnp.zeros((N_PAD, H), jnp.float32)
    ones_rows = jnp.ones((K, H), jnp.float32)

    cnt = _sc_count(dst1, zrow, ones_rows)
    ycat1 = _mm2(x, w1)
    acc1 = _sc_agg(ycat1, adj1, dst1, norm1, zrow)
    ycat2, inv8 = _combine_mm2(acc1, cnt, w2)
    acc2 = _sc_agg(ycat2, adj1, dst1, norm1, zrow)
    wlt = Wl[:H]
    wlb = Wl[H:]
    ws8 = jnp.zeros((H, 8), jnp.float32).at[:, :7].set(Ws)
    bs8 = jnp.full((1, 8), -1e30, jnp.float32).at[0, :7].set(bs)
    out2, lp8 = _final(acc2, inv8, x, wlt, wlb, bl.reshape(1, H), ws8, bs8)

    emotions = jnp.concatenate([x, out2], axis=1)
    return (lp8[:, :7], x, emotions)


# speedup vs baseline: 16.9945x; 1.0343x over previous
import jax
import jax.numpy as jnp
from jax import lax
from jax.experimental import pallas as pl
from jax.experimental.pallas import tpu as pltpu
from jax.experimental.pallas import tpu_sc as plsc

N = 10000
E = 320000
H = 128
NCORE = 2
NSUB = 16
NW = NCORE * NSUB
K = 64
CH = 160
CHA = 240
CHB = 80
NBUF = 4
RING = 8
E_PAD = NW * K * CH
N_PAD = 10240
STRIPE = N_PAD // NSUB
BN = 1000

_MESH = dict(core_axis_name="c", subcore_axis_name="s",
             num_cores=NCORE, num_subcores=NSUB)


def _make_sc_agg():
    def body(ycat, adj1, dst1, norm1, zrow, acc_hbm,
             acc_sh, rows, abuf, dbuf, nbuf, gsm, ssm, asm, dsm, nsm):
        core = lax.axis_index("c")
        sub = lax.axis_index("s")
        heavy = core == 0
        nch = jnp.where(heavy, CHA, CHB)
        base = jnp.where(heavy, sub * CHA * K,
                         NSUB * CHA * K + sub * CHB * K)
        rs = pl.ds(sub * STRIPE, STRIPE)
        pltpu.sync_copy(zrow.at[rs], acc_sh.at[rs])
        plsc.subcore_barrier()

        def issue_edata(c, s):
            off = base + c * K
            pltpu.async_copy(adj1.at[pl.ds(off, K)], abuf[s], asm[s])
            pltpu.async_copy(dst1.at[pl.ds(off, K)], dbuf[s], dsm[s])
            pltpu.async_copy(norm1.at[pl.ds(off, K)], nbuf[s], nsm[s])

        def wait_edata(c, s):
            off = base + c * K
            pltpu.make_async_copy(adj1.at[pl.ds(off, K)], abuf[s],
                                  asm[s]).wait()
            pltpu.make_async_copy(dst1.at[pl.ds(off, K)], dbuf[s],
                                  dsm[s]).wait()
            pltpu.make_async_copy(norm1.at[pl.ds(off, K)], nbuf[s],
                                  nsm[s]).wait()

        def issue_gather(s, p):
            pltpu.async_copy(ycat.at[abuf[s]], rows[p], gsm[p])

        def wait_gather(s, p):
            pltpu.make_async_copy(ycat.at[abuf[s]], rows[p], gsm[p]).wait()

        def issue_scatter(s, p):
            pltpu.async_copy(rows[p], acc_sh.at[dbuf[s]], ssm[p], add=True)

        def wait_scatter(s, p):
            pltpu.make_async_copy(
                rows[p], acc_sh.at[dbuf[s]], ssm[p]).wait()

        def scale(u):
            def group(g, carry2):
                nv = nbuf[u][pl.ds(g * 16, 16)]
                for e16 in range(16):
                    n = nv[e16]
                    row = g * 16 + e16
                    for j in range(H // 16):
                        sl = pl.ds(j * 16, 16)
                        rows[u][row, sl] = rows[u][row, sl] * n
                return carry2
            lax.fori_loop(0, K // 16, group, 0)

        def slot(c, u, e2=True, g1=True, ws=True):
            if e2:
                issue_edata(c + 2, (u + 2) % 4)
            if g1:
                wait_edata(c + 1, (u + 1) % 4)
                issue_gather((u + 1) % 4, (u + 1) % 4)
            wait_gather(u, u)
            scale(u)
            issue_scatter(u, u)
            if ws:
                wait_scatter((u + 3) % 4, (u + 3) % 4)

        issue_edata(0, 0)
        issue_edata(1, 1)
        wait_edata(0, 0)
        issue_gather(0, 0)
        slot(0, 0, ws=False)
        for u in range(1, 4):
            slot(u, u)

        def block(t, carry):
            for u in range(4):
                slot(t * 4 + u, u)
            return carry
        lax.fori_loop(1, nch // 4 - 1, block, 0)

        slot(nch - 4, 0)
        slot(nch - 3, 1)
        slot(nch - 2, 2, e2=False)
        slot(nch - 1, 3, e2=False, g1=False)
        wait_scatter(3, 3)

        plsc.subcore_barrier()
        pltpu.sync_copy(acc_sh.at[rs], acc_hbm.at[core, rs])

    return pl.kernel(
        body,
        out_type=jax.ShapeDtypeStruct((NCORE, N_PAD, H), jnp.float32),
        mesh=plsc.VectorSubcoreMesh(**_MESH),
        scratch_types=[
            pltpu.VMEM_SHARED((N_PAD, H), jnp.float32),
            [pltpu.VMEM((K, H), jnp.float32) for _ in range(NBUF)],
            [pltpu.VMEM((K,), jnp.int32) for _ in range(RING)],
            [pltpu.VMEM((K,), jnp.int32) for _ in range(RING)],
            [pltpu.VMEM((K,), jnp.float32) for _ in range(RING)],
            [pltpu.SemaphoreType.DMA for _ in range(NBUF)],
            [pltpu.SemaphoreType.DMA for _ in range(NBUF)],
            [pltpu.SemaphoreType.DMA for _ in range(RING)],
            [pltpu.SemaphoreType.DMA for _ in range(RING)],
            [pltpu.SemaphoreType.DMA for _ in range(RING)],
        ],
    )


def _make_sc_count():
    def body(dst1, zrow, ones_hbm, cnt_hbm, cnt_sh, dst_v, ones_v):
        core = lax.axis_index("c")
        sub = lax.axis_index("s")
        wid = core * NSUB + sub
        base = wid * CH * K
        rs = pl.ds(sub * STRIPE, STRIPE)
        pltpu.sync_copy(zrow.at[rs], cnt_sh.at[rs])
        pltpu.sync_copy(ones_hbm, ones_v)
        plsc.subcore_barrier()

        def chunk(c, carry):
            off = base + c * K
            pltpu.sync_copy(dst1.at[pl.ds(off, K)], dst_v)
            pltpu.sync_copy(ones_v, cnt_sh.at[dst_v], add=True)
            return carry
        lax.fori_loop(0, CH, chunk, 0)

        plsc.subcore_barrier()
        pltpu.sync_copy(cnt_sh.at[rs], cnt_hbm.at[core, rs])

    return pl.kernel(
        body,
        out_type=jax.ShapeDtypeStruct((NCORE, N_PAD, H), jnp.float32),
        mesh=plsc.VectorSubcoreMesh(**_MESH),
        scratch_types=[
            pltpu.VMEM_SHARED((N_PAD, H), jnp.float32),
            pltpu.VMEM((K,), jnp.int32),
            pltpu.VMEM((K, H), jnp.float32),
        ],
    )


def _mm2_body(x_ref, w_ref, o_ref):
    o_ref[...] = jnp.dot(x_ref[...], w_ref[0],
                         preferred_element_type=jnp.float32)


def _mm2(x, w):
    return pl.pallas_call(
        _mm2_body,
        grid=(2, N // BN),
        in_specs=[
            pl.BlockSpec((BN, H), lambda r, i: (i, 0)),
            pl.BlockSpec((1, H, H), lambda r, i: (r, 0, 0)),
        ],
        out_specs=pl.BlockSpec((BN, H), lambda r, i: (r * (N // BN) + i, 0)),
        out_shape=jax.ShapeDtypeStruct((2 * N, H), jnp.float32),
    )(x, w)


def _combine_mm2_body(acc_ref, cnt_ref, w_ref, o_ref, inv_ref):
    ssum = acc_ref[0] + acc_ref[1]
    cnt = cnt_ref[0, :, 0] + cnt_ref[1, :, 0]
    inv = 1.0 / jnp.maximum(cnt, 1.0)
    out1 = jax.nn.sigmoid(ssum * inv[:, None])
    o_ref[...] = jnp.dot(out1, w_ref[0], preferred_element_type=jnp.float32)
    inv_ref[...] = inv[:, None] * jnp.ones((1, 8), jnp.float32)


def _combine_mm2(acc, cnt, w):
    return pl.pallas_call(
        _combine_mm2_body,
        grid=(2, N // BN),
        in_specs=[
            pl.BlockSpec((2, BN, H), lambda r, i: (0, i, 0)),
            pl.BlockSpec((2, BN, H), lambda r, i: (0, i, 0)),
            pl.BlockSpec((1, H, H), lambda r, i: (r, 0, 0)),
        ],
        out_specs=[
            pl.BlockSpec((BN, H), lambda r, i: (r * (N // BN) + i, 0)),
            pl.BlockSpec((BN, 8), lambda r, i: (i, 0)),
        ],
        out_shape=[
            jax.ShapeDtypeStruct((2 * N, H), jnp.float32),
            jax.ShapeDtypeStruct((N, 8), jnp.float32),
        ],
    )(acc, cnt, w)


def _final_body(acc_ref, inv_ref, x_ref, wlt_ref, wlb_ref, bl_ref,
                ws_ref, bs_ref, out2_ref, lp_ref):
    inv = inv_ref[:, 0]
    out2 = (acc_ref[0] + acc_ref[1]) * inv[:, None]
    out2_ref[...] = out2
    x = x_ref[...]
    hidden = jnp.dot(x, wlt_ref[...], preferred_element_type=jnp.float32)
    hidden += jnp.dot(out2, wlb_ref[...], preferred_element_type=jnp.float32)
    hidden = jax.nn.relu(hidden + bl_ref[...])
    logits = jnp.dot(hidden, ws_ref[...],
                     preferred_element_type=jnp.float32) + bs_ref[...]
    m = jnp.max(logits, axis=1, keepdims=True)
    z = logits - m
    lse = jnp.log(jnp.sum(jnp.exp(z), axis=1, keepdims=True))
    lp_ref[...] = z - lse


def _final(acc, inv8, x, wlt, wlb, bl, ws8, bs8):
    return pl.pallas_call(
        _final_body,
        grid=(N // BN,),
        in_specs=[
            pl.BlockSpec((2, BN, H), lambda i: (0, i, 0)),
            pl.BlockSpec((BN, 8), lambda i: (i, 0)),
            pl.BlockSpec((BN, H), lambda i: (i, 0)),
            pl.BlockSpec((H, H), lambda i: (0, 0)),
            pl.BlockSpec((H, H), lambda i: (0, 0)),
            pl.BlockSpec((1, H), lambda i: (0, 0)),
            pl.BlockSpec((H, 8), lambda i: (0, 0)),
            pl.BlockSpec((1, 8), lambda i: (0, 0)),
        ],
        out_specs=[
            pl.BlockSpec((BN, H), lambda i: (i, 0)),
            pl.BlockSpec((BN, 8), lambda i: (i, 0)),
        ],
        out_shape=[
            jax.ShapeDtypeStruct((N, H), jnp.float32),
            jax.ShapeDtypeStruct((N, 8), jnp.float32),
        ],
    )(acc, inv8, x, wlt, wlb, bl, ws8, bs8)


_sc_agg = _make_sc_agg()
_sc_count = _make_sc_count()


@jax.jit
def kernel(speaker, x, edge_index, edge_norm, edge_type, seq_lengths, umask,
           w1, w2, Wl, bl, Ws, bs):
    dst = edge_index[0]
    src = edge_index[1]
    adj = edge_type * N + src
    norm2 = edge_norm * 2.0

    pad = E_PAD - E
    adj1 = jnp.concatenate([adj, jnp.zeros((pad,), jnp.int32)])
    dst1 = jnp.concatenate([dst, jnp.full((pad,), N, jnp.int32)])
    norm1 = jnp.concatenate([norm2, jnp.zeros((pad,), jnp.float32)])

    zrow = jnp.zeros((N_PAD, H), jnp.float32)
    ones_rows = jnp.ones((K, H), jnp.float32)

    cnt = _sc_count(dst1, zrow, ones_rows)
    ycat1 = _mm2(x, w1)
    acc1 = _sc_agg(ycat1, adj1, dst1, norm1, zrow)
    ycat2, inv8 = _combine_mm2(acc1, cnt, w2)
    acc2 = _sc_agg(ycat2, adj1, dst1, norm1, zrow)
    wlt = Wl[:H]
    wlb = Wl[H:]
    ws8 = jnp.zeros((H, 8), jnp.float32).at[:, :7].set(Ws)
    bs8 = jnp.full((1, 8), -1e30, jnp.float32).at[0, :7].set(bs)
    out2, lp8 = _final(acc2, inv8, x, wlt, wlb, bl.reshape(1, H), ws8, bs8)

    emotions = jnp.concatenate([x, out2], axis=1)
    return (lp8[:, :7], x, emotions)


# speedup vs baseline: 17.2487x; 1.0150x over previous
import jax
import jax.numpy as jnp
from jax import lax
from jax.experimental import pallas as pl
from jax.experimental.pallas import tpu as pltpu
from jax.experimental.pallas import tpu_sc as plsc

N = 10000
E = 320000
H = 128
NCORE = 2
NSUB = 16
NW = NCORE * NSUB
K = 64
CH = 160
CHA = 240
CHB = 80
NBUF = 4
RING = 8
E_PAD = NW * K * CH
N_PAD = 10240
STRIPE = N_PAD // NSUB
BN = 1000

_MESH = dict(core_axis_name="c", subcore_axis_name="s",
             num_cores=NCORE, num_subcores=NSUB)


def _make_sc_agg():
    def body(ycat, adj1, dst1, norm1, zrow, acc_hbm,
             acc_sh, rows, abuf, dbuf, nbuf, gsm, ssm, asm, dsm, nsm):
        core = lax.axis_index("c")
        sub = lax.axis_index("s")
        heavy = core == 1
        nch = jnp.where(heavy, CHA, CHB)
        base = jnp.where(heavy, sub * CHA * K,
                         NSUB * CHA * K + sub * CHB * K)
        rs = pl.ds(sub * STRIPE, STRIPE)
        pltpu.sync_copy(zrow.at[rs], acc_sh.at[rs])
        plsc.subcore_barrier()

        def issue_edata(c, s):
            off = base + c * K
            pltpu.async_copy(adj1.at[pl.ds(off, K)], abuf[s], asm[s])
            pltpu.async_copy(dst1.at[pl.ds(off, K)], dbuf[s], dsm[s])
            pltpu.async_copy(norm1.at[pl.ds(off, K)], nbuf[s], nsm[s])

        def wait_edata(c, s):
            off = base + c * K
            pltpu.make_async_copy(adj1.at[pl.ds(off, K)], abuf[s],
                                  asm[s]).wait()
            pltpu.make_async_copy(dst1.at[pl.ds(off, K)], dbuf[s],
                                  dsm[s]).wait()
            pltpu.make_async_copy(norm1.at[pl.ds(off, K)], nbuf[s],
                                  nsm[s]).wait()

        def issue_gather(s, p):
            pltpu.async_copy(ycat.at[abuf[s]], rows[p], gsm[p])

        def wait_gather(s, p):
            pltpu.make_async_copy(ycat.at[abuf[s]], rows[p], gsm[p]).wait()

        def issue_scatter(s, p):
            pltpu.async_copy(rows[p], acc_sh.at[dbuf[s]], ssm[p], add=True)

        def wait_scatter(s, p):
            pltpu.make_async_copy(
                rows[p], acc_sh.at[dbuf[s]], ssm[p]).wait()

        def scale(u):
            def group(g, carry2):
                nv = nbuf[u][pl.ds(g * 16, 16)]
                for e16 in range(16):
                    n = nv[e16]
                    row = g * 16 + e16
                    for j in range(H // 16):
                        sl = pl.ds(j * 16, 16)
                        rows[u][row, sl] = rows[u][row, sl] * n
                return carry2
            lax.fori_loop(0, K // 16, group, 0)

        def slot(c, u, e2=True, g1=True, ws=True):
            if e2:
                issue_edata(c + 2, (u + 2) % 4)
            if g1:
                wait_edata(c + 1, (u + 1) % 4)
                issue_gather((u + 1) % 4, (u + 1) % 4)
            wait_gather(u, u)
            scale(u)
            issue_scatter(u, u)
            if ws:
                wait_scatter((u + 3) % 4, (u + 3) % 4)

        issue_edata(0, 0)
        issue_edata(1, 1)
        wait_edata(0, 0)
        issue_gather(0, 0)
        slot(0, 0, ws=False)
        for u in range(1, 4):
            slot(u, u)

        def block(t, carry):
            for u in range(4):
                slot(t * 4 + u, u)
            return carry
        lax.fori_loop(1, nch // 4 - 1, block, 0)

        slot(nch - 4, 0)
        slot(nch - 3, 1)
        slot(nch - 2, 2, e2=False)
        slot(nch - 1, 3, e2=False, g1=False)
        wait_scatter(3, 3)

        plsc.subcore_barrier()
        pltpu.sync_copy(acc_sh.at[rs], acc_hbm.at[core, rs])

    return pl.kernel(
        body,
        out_type=jax.ShapeDtypeStruct((NCORE, N_PAD, H), jnp.float32),
        mesh=plsc.VectorSubcoreMesh(**_MESH),
        scratch_types=[
            pltpu.VMEM_SHARED((N_PAD, H), jnp.float32),
            [pltpu.VMEM((K, H), jnp.float32) for _ in range(NBUF)],
            [pltpu.VMEM((K,), jnp.int32) for _ in range(RING)],
            [pltpu.VMEM((K,), jnp.int32) for _ in range(RING)],
            [pltpu.VMEM((K,), jnp.float32) for _ in range(RING)],
            [pltpu.SemaphoreType.DMA for _ in range(NBUF)],
            [pltpu.SemaphoreType.DMA for _ in range(NBUF)],
            [pltpu.SemaphoreType.DMA for _ in range(RING)],
            [pltpu.SemaphoreType.DMA for _ in range(RING)],
            [pltpu.SemaphoreType.DMA for _ in range(RING)],
        ],
    )


def _make_sc_count():
    def body(dst1, zrow, ones_hbm, cnt_hbm, cnt_sh, dst_v, ones_v):
        core = lax.axis_index("c")
        sub = lax.axis_index("s")
        wid = core * NSUB + sub
        base = wid * CH * K
        rs = pl.ds(sub * STRIPE, STRIPE)
        pltpu.sync_copy(zrow.at[rs], cnt_sh.at[rs])
        pltpu.sync_copy(ones_hbm, ones_v)
        plsc.subcore_barrier()

        def chunk(c, carry):
            off = base + c * K
            pltpu.sync_copy(dst1.at[pl.ds(off, K)], dst_v)
            pltpu.sync_copy(ones_v, cnt_sh.at[dst_v], add=True)
            return carry
        lax.fori_loop(0, CH, chunk, 0)

        plsc.subcore_barrier()
        pltpu.sync_copy(cnt_sh.at[rs], cnt_hbm.at[core, rs])

    return pl.kernel(
        body,
        out_type=jax.ShapeDtypeStruct((NCORE, N_PAD, H), jnp.float32),
        mesh=plsc.VectorSubcoreMesh(**_MESH),
        scratch_types=[
            pltpu.VMEM_SHARED((N_PAD, H), jnp.float32),
            pltpu.VMEM((K,), jnp.int32),
            pltpu.VMEM((K, H), jnp.float32),
        ],
    )


def _mm2_body(x_ref, w_ref, o_ref):
    o_ref[...] = jnp.dot(x_ref[...], w_ref[0],
                         preferred_element_type=jnp.float32)


def _mm2(x, w):
    return pl.pallas_call(
        _mm2_body,
        grid=(2, N // BN),
        in_specs=[
            pl.BlockSpec((BN, H), lambda r, i: (i, 0)),
            pl.BlockSpec((1, H, H), lambda r, i: (r, 0, 0)),
        ],
        out_specs=pl.BlockSpec((BN, H), lambda r, i: (r * (N // BN) + i, 0)),
        out_shape=jax.ShapeDtypeStruct((2 * N, H), jnp.float32),
    )(x, w)


def _combine_mm2_body(acc_ref, cnt_ref, w_ref, o_ref, inv_ref):
    ssum = acc_ref[0] + acc_ref[1]
    cnt = cnt_ref[0, :, 0] + cnt_ref[1, :, 0]
    inv = 1.0 / jnp.maximum(cnt, 1.0)
    out1 = jax.nn.sigmoid(ssum * inv[:, None])
    o_ref[...] = jnp.dot(out1, w_ref[0], preferred_element_type=jnp.float32)
    inv_ref[...] = inv[:, None] * jnp.ones((1, 8), jnp.float32)


def _combine_mm2(acc, cnt, w):
    return pl.pallas_call(
        _combine_mm2_body,
        grid=(2, N // BN),
        in_specs=[
            pl.BlockSpec((2, BN, H), lambda r, i: (0, i, 0)),
            pl.BlockSpec((2, BN, H), lambda r, i: (0, i, 0)),
            pl.BlockSpec((1, H, H), lambda r, i: (r, 0, 0)),
        ],
        out_specs=[
            pl.BlockSpec((BN, H), lambda r, i: (r * (N // BN) + i, 0)),
            pl.BlockSpec((BN, 8), lambda r, i: (i, 0)),
        ],
        out_shape=[
            jax.ShapeDtypeStruct((2 * N, H), jnp.float32),
            jax.ShapeDtypeStruct((N, 8), jnp.float32),
        ],
    )(acc, cnt, w)


def _final_body(acc_ref, inv_ref, x_ref, wlt_ref, wlb_ref, bl_ref,
                ws_ref, bs_ref, out2_ref, lp_ref):
    inv = inv_ref[:, 0]
    out2 = (acc_ref[0] + acc_ref[1]) * inv[:, None]
    out2_ref[...] = out2
    x = x_ref[...]
    hidden = jnp.dot(x, wlt_ref[...], preferred_element_type=jnp.float32)
    hidden += jnp.dot(out2, wlb_ref[...], preferred_element_type=jnp.float32)
    hidden = jax.nn.relu(hidden + bl_ref[...])
    logits = jnp.dot(hidden, ws_ref[...],
                     preferred_element_type=jnp.float32) + bs_ref[...]
    m = jnp.max(logits, axis=1, keepdims=True)
    z = logits - m
    lse = jnp.log(jnp.sum(jnp.exp(z), axis=1, keepdims=True))
    lp_ref[...] = z - lse


def _final(acc, inv8, x, wlt, wlb, bl, ws8, bs8):
    return pl.pallas_call(
        _final_body,
        grid=(N // BN,),
        in_specs=[
            pl.BlockSpec((2, BN, H), lambda i: (0, i, 0)),
            pl.BlockSpec((BN, 8), lambda i: (i, 0)),
            pl.BlockSpec((BN, H), lambda i: (i, 0)),
            pl.BlockSpec((H, H), lambda i: (0, 0)),
            pl.BlockSpec((H, H), lambda i: (0, 0)),
            pl.BlockSpec((1, H), lambda i: (0, 0)),
            pl.BlockSpec((H, 8), lambda i: (0, 0)),
            pl.BlockSpec((1, 8), lambda i: (0, 0)),
        ],
        out_specs=[
            pl.BlockSpec((BN, H), lambda i: (i, 0)),
            pl.BlockSpec((BN, 8), lambda i: (i, 0)),
        ],
        out_shape=[
            jax.ShapeDtypeStruct((N, H), jnp.float32),
            jax.ShapeDtypeStruct((N, 8), jnp.float32),
        ],
    )(acc, inv8, x, wlt, wlb, bl, ws8, bs8)


_sc_agg = _make_sc_agg()
_sc_count = _make_sc_count()


@jax.jit
def kernel(speaker, x, edge_index, edge_norm, edge_type, seq_lengths, umask,
           w1, w2, Wl, bl, Ws, bs):
    dst = edge_index[0]
    src = edge_index[1]
    adj = edge_type * N + src
    norm2 = edge_norm * 2.0

    pad = E_PAD - E
    adj1 = jnp.concatenate([adj, jnp.zeros((pad,), jnp.int32)])
    dst1 = jnp.concatenate([dst, jnp.full((pad,), N, jnp.int32)])
    norm1 = jnp.concatenate([norm2, jnp.zeros((pad,), jnp.float32)])

    zrow = jnp.zeros((N_PAD, H), jnp.float32)
    ones_rows = jnp.ones((K, H), jnp.float32)

    cnt = _sc_count(dst1, zrow, ones_rows)
    ycat1 = _mm2(x, w1)
    acc1 = _sc_agg(ycat1, adj1, dst1, norm1, zrow)
    ycat2, inv8 = _combine_mm2(acc1, cnt, w2)
    acc2 = _sc_agg(ycat2, adj1, dst1, norm1, zrow)
    wlt = Wl[:H]
    wlb = Wl[H:]
    ws8 = jnp.zeros((H, 8), jnp.float32).at[:, :7].set(Ws)
    bs8 = jnp.full((1, 8), -1e30, jnp.float32).at[0, :7].set(bs)
    out2, lp8 = _final(acc2, inv8, x, wlt, wlb, bl.reshape(1, H), ws8, bs8)

    emotions = jnp.concatenate([x, out2], axis=1)
    return (lp8[:, :7], x, emotions)
